# baseline (speedup 1.0000x reference)
import jax
import jax.numpy as jnp
from jax import lax
from jax.experimental import pallas as pl
from jax.experimental.pallas import tpu as pltpu

S = 2048
DIM = 2048
NH, HD, RD = 16, 128, 64
QLR = 1536
HALF = RD // 2
f32 = jnp.float32
bf16 = jnp.bfloat16
i32 = jnp.int32
DN = (((1,), (1,)), ((), ()))

_SCALE1 = NH ** -0.5
_SCALE2 = HD ** -0.5


def _rope_mix(v, cosb, sinb):
    R, N = v.shape
    reps = N // 128
    ones = jnp.ones((R, 64), f32)
    zeros = jnp.zeros((R, 64), f32)
    a_patt = jnp.concatenate([cosb, cosb, ones], axis=1)
    b_patt = jnp.concatenate([-sinb, sinb, zeros], axis=1)
    if reps > 1:
        a_patt = jnp.concatenate([a_patt] * reps, axis=1)
        b_patt = jnp.concatenate([b_patt] * reps, axis=1)
    lane = lax.broadcasted_iota(i32, (R, N), 1) % 128
    fwd = pltpu.roll(v, N - HALF, 1)
    bwd = pltpu.roll(v, HALF, 1)
    partner = jnp.where(lane < HALF, fwd, bwd)
    mixed = v * a_patt + partner * b_patt
    return jnp.where(lane < RD, mixed, v)


def _kw_body(x_ref, wk_ref, wp_ref, knw_ref, knb_ref, cos_ref, sin_ref,
             k_out, w_out):
    xb = x_ref[...]
    k = lax.dot_general(xb, wk_ref[...], DN, preferred_element_type=f32)
    mu = jnp.mean(k, axis=-1, keepdims=True)
    var = jnp.var(k, axis=-1, keepdims=True)
    k = (k - mu) / jnp.sqrt(var + 1e-6) * knw_ref[...] + knb_ref[...]
    k_out[...] = _rope_mix(k, cos_ref[...], sin_ref[...])
    w = lax.dot_general(wp_ref[...], xb, DN, preferred_element_type=f32)
    w_out[...] = w * _SCALE1 * _SCALE2


def _q_body(qr_ref, wq_ref, cos_ref, sin_ref, q_out):
    q = lax.dot_general(qr_ref[...], wq_ref[...], DN, preferred_element_type=f32)
    q_out[...] = _rope_mix(q, cos_ref[...], sin_ref[...])


LB = 512


def _logits_body(k_ref, q_ref, o_ref):
    lg = lax.dot_general(k_ref[...], q_ref[...], DN,
                         preferred_element_type=f32)
    o_ref[...] = jnp.maximum(lg, 0.0).astype(bf16)


SB = 128


def _hsum_body(l_ref, w_ref, o_ref):
    lgb = l_ref[...]
    wv = w_ref[...].astype(bf16)
    bt = jnp.broadcast_to(wv.T[None], (SB, NH, SB)).reshape(SB * NH, SB)
    kk = lax.broadcasted_iota(i32, (SB * NH, SB), 0)
    jj = lax.broadcasted_iota(i32, (SB * NH, SB), 1)
    wbd = jnp.where(kk // NH == jj, bt, jnp.zeros_like(bt))
    o_ref[...] = lax.dot_general(lgb, wbd, (((1,), (0,)), ((), ())),
                                 preferred_element_type=f32)


CB = 128


def _sort_body(s_ref, o_ref):
    v = s_ref[...]
    b = lax.bitcast_convert_type(v, i32)
    key0 = jnp.where(b < 0, b ^ jnp.int32(0x7FFFFFFF), b)
    key0 = jnp.where(v == 0.0, jnp.int32(0), key0)
    idx0 = lax.broadcasted_iota(i32, (S, CB), 0)
    it = lax.broadcasted_iota(i32, (S, CB), 0)

    def step(state):
        key, idx, kk, j = state
        is_first = (it & j) == 0
        want_larger = is_first == ((it & kk) == 0)
        pk = jnp.where(is_first, pltpu.roll(key, S - j, 0),
                       pltpu.roll(key, j, 0))
        pi = jnp.where(is_first, pltpu.roll(idx, S - j, 0),
                       pltpu.roll(idx, j, 0))
        eq = pk == key
        first = (pk > key) | (eq & (pi < idx))
        last = (pk < key) | (eq & (pi > idx))
        take = (want_larger & first) | (~want_larger & last)
        key = jnp.where(take, pk, key)
        idx = jnp.where(take, pi, idx)
        last_inner = j == 1
        kk_n = jnp.where(last_inner, kk * 2, kk)
        j_n = jnp.where(last_inner, kk, j // 2)
        return key, idx, kk_n, j_n

    key, idx, _, _ = lax.while_loop(
        lambda st: st[2] <= S,
        step,
        (key0, idx0, jnp.int32(2), jnp.int32(1)),
    )
    o_ref[...] = idx


@jax.jit
def kernel(x, qr, start_pos, freqs_cis, mask, wq_b_w, wk_w, knorm_w, knorm_b,
           wproj_w):
    x2 = x[0]
    qr2 = qr[0]
    cosT = jnp.cos(freqs_cis)
    sinT = jnp.sin(freqs_cis)

    BK = S
    kf, wgt = pl.pallas_call(
        _kw_body,
        grid=(S // BK,),
        in_specs=[pl.BlockSpec((BK, DIM), lambda i: (i, 0)),
                  pl.BlockSpec((HD, DIM), lambda i: (0, 0)),
                  pl.BlockSpec((NH, DIM), lambda i: (0, 0)),
                  pl.BlockSpec((1, HD), lambda i: (0, 0)),
                  pl.BlockSpec((1, HD), lambda i: (0, 0)),
                  pl.BlockSpec((BK, HALF), lambda i: (i, 0)),
                  pl.BlockSpec((BK, HALF), lambda i: (i, 0))],
        out_specs=[pl.BlockSpec((BK, HD), lambda i: (i, 0)),
                   pl.BlockSpec((NH, BK), lambda i: (0, i))],
        out_shape=[jax.ShapeDtypeStruct((S, HD), f32),
                   jax.ShapeDtypeStruct((NH, S), f32)],
    )(x2, wk_w, wproj_w, knorm_w.reshape(1, HD), knorm_b.reshape(1, HD),
      cosT, sinT)
    wgt = wgt.T

    BS = 512
    qf = pl.pallas_call(
        _q_body,
        grid=(S // BS,),
        in_specs=[pl.BlockSpec((BS, QLR), lambda i: (i, 0)),
                  pl.BlockSpec((NH * HD, QLR), lambda i: (0, 0)),
                  pl.BlockSpec((BS, HALF), lambda i: (i, 0)),
                  pl.BlockSpec((BS, HALF), lambda i: (i, 0))],
        out_specs=pl.BlockSpec((BS, NH * HD), lambda i: (i, 0)),
        out_shape=jax.ShapeDtypeStruct((S, NH * HD), f32),
    )(qr2, wq_b_w, cosT, sinT)

    q_rs = qf.reshape(S * NH, HD)

    lgT = pl.pallas_call(
        _logits_body,
        grid=(S * NH // LB,),
        in_specs=[pl.BlockSpec((S, HD), lambda i: (0, 0)),
                  pl.BlockSpec((LB, HD), lambda i: (i, 0))],
        out_specs=pl.BlockSpec((S, LB), lambda i: (0, i)),
        out_shape=jax.ShapeDtypeStruct((S, S * NH), bf16),
    )(kf, q_rs)

    scT = pl.pallas_call(
        _hsum_body,
        grid=(S // SB,),
        in_specs=[pl.BlockSpec((S, SB * NH), lambda i: (0, i)),
                  pl.BlockSpec((SB, NH), lambda i: (i, 0))],
        out_specs=pl.BlockSpec((S, SB), lambda i: (0, i)),
        out_shape=jax.ShapeDtypeStruct((S, S), f32),
    )(lgT, wgt)

    idxT = pl.pallas_call(
        _sort_body,
        grid=(S // CB,),
        in_specs=[pl.BlockSpec((S, CB), lambda i: (0, i))],
        out_specs=pl.BlockSpec((S, CB), lambda i: (0, i)),
        out_shape=jax.ShapeDtypeStruct((S, S), i32),
    )(scT)

    return idxT.T[None]
